# baseline (speedup 1.0000x reference)
import jax
import jax.numpy as jnp
from jax.experimental import pallas as pl

B = 2
D = 64
C = 16
S = 3


def _shift_fold(t, d, axis):
    if d == 0:
        return t

    def sl(a, b):
        idx = [slice(None)] * t.ndim
        idx[axis] = slice(a, b)
        return t[tuple(idx)]

    zshape = list(t.shape)
    zshape[axis] = 1
    zero = jnp.zeros(zshape, t.dtype)
    if d == 1:
        return jnp.concatenate([zero, sl(0, D - 2), sl(D - 2, D - 1) + sl(D - 1, D)], axis=axis)
    else:
        return jnp.concatenate([sl(0, 1) + sl(1, 2), sl(2, D), zero], axis=axis)


def _conv_body(prev_ref, cur_ref, next_ref, kz_ref, m_ref, b_ref, out_ref):
    i = pl.program_id(0)
    z = i % D
    k0 = kz_ref[0]
    k1 = kz_ref[1]
    k2 = kz_ref[2]
    kp = jnp.where(z == 0, k0, k2)
    kn = jnp.where(z == D - 1, k2, k0)
    p3 = jnp.concatenate([prev_ref[0], cur_ref[0], next_ref[0]], axis=1)
    k3 = jnp.concatenate([kp, k1, kn], axis=0)
    v = jnp.dot(p3, k3, preferred_element_type=jnp.float32)
    v = v.reshape(D, D, 9 * C)
    acc = None
    for ky in range(S):
        for kx in range(S):
            c0 = (ky * S + kx) * C
            t = v[:, :, c0:c0 + C]
            t = _shift_fold(t, ky - 1, 0)
            t = _shift_fold(t, kx - 1, 1)
            acc = t if acc is None else acc + t
    m = m_ref[0]
    bias = b_ref[0]
    res = (acc.reshape(D * D, C) + m * bias) * m
    out_ref[0] = res


def kernel(in_idx_b, in_idx_sp, in_vals, mask_idx_b, mask_idx_sp, mask_vals, kernel, bias):
    nv = B * D * D * D
    fi = ((in_idx_b * D + in_idx_sp[0]) * D + in_idx_sp[1]) * D + in_idx_sp[2]
    fm = ((mask_idx_b * D + mask_idx_sp[0]) * D + mask_idx_sp[1]) * D + mask_idx_sp[2]
    idx = jnp.concatenate([fi, fm + nv])
    upd = jnp.concatenate([in_vals, mask_vals], axis=0)
    gm = jnp.zeros((2 * nv, C), jnp.float32).at[idx].add(upd)
    gm = gm.reshape(2 * B * D, D * D, C)

    kz = kernel.transpose(0, 3, 1, 2, 4).reshape(S, C, S * S * C)
    bias2 = bias.reshape(1, C)

    plane = pl.BlockSpec((1, D * D, C), lambda i: (i, 0, 0))
    out = pl.pallas_call(
        _conv_body,
        grid=(B * D,),
        in_specs=[
            pl.BlockSpec((1, D * D, C), lambda i: (jnp.where(i % D > 0, i - 1, i), 0, 0)),
            plane,
            pl.BlockSpec((1, D * D, C), lambda i: (jnp.where(i % D < D - 1, i + 1, i), 0, 0)),
            pl.BlockSpec((S, C, S * S * C), lambda i: (0, 0, 0)),
            pl.BlockSpec((1, D * D, C), lambda i: (B * D + i, 0, 0)),
            pl.BlockSpec((1, C), lambda i: (0, 0)),
        ],
        out_specs=plane,
        out_shape=jax.ShapeDtypeStruct((B * D, D * D, C), jnp.float32),
    )(gm, gm, gm, kz, gm, bias2)
    return out.reshape(B, D, D, D, C)


# speedup vs baseline: 1.0906x; 1.0906x over previous
import jax
import jax.numpy as jnp
from jax.experimental import pallas as pl

B = 2
D = 64
C = 16
S = 3


def _shift_fold(t, d, axis):
    if d == 0:
        return t

    def sl(a, b):
        idx = [slice(None)] * t.ndim
        idx[axis] = slice(a, b)
        return t[tuple(idx)]

    zshape = list(t.shape)
    zshape[axis] = 1
    zero = jnp.zeros(zshape, t.dtype)
    if d == 1:
        return jnp.concatenate([zero, sl(0, D - 2), sl(D - 2, D - 1) + sl(D - 1, D)], axis=axis)
    else:
        return jnp.concatenate([sl(0, 1) + sl(1, 2), sl(2, D), zero], axis=axis)


def _conv_body(prev_ref, cur_ref, next_ref, kz_ref, m_ref, b_ref, out_ref):
    i = pl.program_id(0)
    z = i % D
    k0 = kz_ref[0]
    k1 = kz_ref[1]
    k2 = kz_ref[2]
    kp = jnp.where(z == 0, k0, k2)
    kn = jnp.where(z == D - 1, k2, k0)
    p3 = jnp.concatenate([prev_ref[0], cur_ref[0], next_ref[0]], axis=1)
    k3 = jnp.concatenate([kp, k1, kn], axis=0)
    v = jnp.dot(p3, k3, preferred_element_type=jnp.float32)
    v = v.reshape(D, D, 9 * C)
    acc = None
    for ky in range(S):
        for kx in range(S):
            c0 = (ky * S + kx) * C
            t = v[:, :, c0:c0 + C]
            t = _shift_fold(t, ky - 1, 0)
            t = _shift_fold(t, kx - 1, 1)
            acc = t if acc is None else acc + t
    m = m_ref[0]
    bias = b_ref[0]
    res = (acc.reshape(D * D, C) + m * bias) * m
    out_ref[0] = res


def kernel(in_idx_b, in_idx_sp, in_vals, mask_idx_b, mask_idx_sp, mask_vals, kernel, bias):
    fi = ((in_idx_b * D + in_idx_sp[0]) * D + in_idx_sp[1]) * D + in_idx_sp[2]
    g = jnp.zeros((B * D * D * D, C), jnp.float32).at[fi].add(in_vals)
    fm = ((mask_idx_b * D + mask_idx_sp[0]) * D + mask_idx_sp[1]) * D + mask_idx_sp[2]
    m = jnp.zeros((B * D * D * D, C), jnp.float32).at[fm].add(mask_vals)
    g = g.reshape(B * D, D * D, C)
    m = m.reshape(B * D, D * D, C)

    kz = kernel.transpose(0, 3, 1, 2, 4).reshape(S, C, S * S * C)
    bias2 = bias.reshape(1, C)

    plane = pl.BlockSpec((1, D * D, C), lambda i: (i, 0, 0))
    out = pl.pallas_call(
        _conv_body,
        grid=(B * D,),
        in_specs=[
            pl.BlockSpec((1, D * D, C), lambda i: (jnp.where(i % D > 0, i - 1, i), 0, 0)),
            plane,
            pl.BlockSpec((1, D * D, C), lambda i: (jnp.where(i % D < D - 1, i + 1, i), 0, 0)),
            pl.BlockSpec((S, C, S * S * C), lambda i: (0, 0, 0)),
            plane,
            pl.BlockSpec((1, C), lambda i: (0, 0)),
        ],
        out_specs=plane,
        out_shape=jax.ShapeDtypeStruct((B * D, D * D, C), jnp.float32),
    )(g, g, g, kz, m, bias2)
    return out.reshape(B, D, D, D, C)
